# baseline (speedup 1.0000x reference)
import functools

import jax
import jax.numpy as jnp
from jax import lax
from jax.experimental import pallas as pl
from jax.experimental.pallas import tpu as pltpu
from jax.experimental.pallas import tpu_sc as plsc

VOCAB = 100000
HIDDEN = 128
B = 1024
L = 50

NC = 2
NS = 16
NW = NC * NS

TOTAL = B * L
PER_W = TOTAL // NW
CHUNK = 400
N_CHUNKS = PER_W // CHUNK

SEQ_PER_TC_BLOCK = 4


def _sc_gather(ids_flat, token_embedding):
    mesh = plsc.VectorSubcoreMesh(core_axis_name="c", subcore_axis_name="s")

    @functools.partial(
        pl.kernel,
        out_type=jax.ShapeDtypeStruct((TOTAL, HIDDEN), jnp.float32),
        mesh=mesh,
        scratch_types=[
            pltpu.VMEM((PER_W,), jnp.int32),
            pltpu.VMEM((CHUNK, HIDDEN), jnp.float32),
            pltpu.VMEM((CHUNK, HIDDEN), jnp.float32),
            pltpu.SemaphoreType.DMA,
            pltpu.SemaphoreType.DMA,
            pltpu.SemaphoreType.DMA,
            pltpu.SemaphoreType.DMA,
        ],
    )
    def k(ids_hbm, table_hbm, out_hbm,
          idx_v, rows0, rows1, gsem0, gsem1, osem0, osem1):
        wid = lax.axis_index("s") * NC + lax.axis_index("c")
        base = wid * PER_W
        pltpu.sync_copy(ids_hbm.at[pl.ds(base, PER_W)], idx_v)

        rows = (rows0, rows1)
        gsems = (gsem0, gsem1)
        osems = (osem0, osem1)

        def start_gather(g):
            return pltpu.async_copy(
                table_hbm.at[idx_v.at[pl.ds(g * CHUNK, CHUNK)]],
                rows[g % 2], gsems[g % 2])

        def start_out(g):
            return pltpu.async_copy(
                rows[g % 2], out_hbm.at[pl.ds(base + g * CHUNK, CHUNK)],
                osems[g % 2])

        gcp = [None] * N_CHUNKS
        ocp = [None] * N_CHUNKS
        gcp[0] = start_gather(0)
        for g in range(N_CHUNKS):
            if g + 1 < N_CHUNKS:
                if g + 1 >= 2:
                    ocp[g - 1].wait()
                gcp[g + 1] = start_gather(g + 1)
            gcp[g].wait()
            ocp[g] = start_out(g)
        ocp[N_CHUNKS - 2].wait()
        ocp[N_CHUNKS - 1].wait()

    return k(ids_flat, token_embedding)


def _tc_add_body(tok_ref, pos_ref, out_ref):
    p = pos_ref[...]
    for s in range(SEQ_PER_TC_BLOCK):
        out_ref[s, :, :] = tok_ref[pl.ds(s * L, L), :] + p


def _tc_add(tok_flat, pos_block):
    grid = (B // SEQ_PER_TC_BLOCK,)
    return pl.pallas_call(
        _tc_add_body,
        grid=grid,
        in_specs=[
            pl.BlockSpec((SEQ_PER_TC_BLOCK * L, HIDDEN), lambda i: (i, 0)),
            pl.BlockSpec((L, HIDDEN), lambda i: (0, 0)),
        ],
        out_specs=pl.BlockSpec((SEQ_PER_TC_BLOCK, L, HIDDEN),
                               lambda i: (i, 0, 0)),
        out_shape=jax.ShapeDtypeStruct((B, L, HIDDEN), jnp.float32),
    )(tok_flat, pos_block)


def kernel(input_ids, token_embedding, position_embedding):
    ids_flat = input_ids.reshape(TOTAL).astype(jnp.int32)
    pos_block = position_embedding[0, :L, :]
    tok_flat = _sc_gather(ids_flat, token_embedding)
    return _tc_add(tok_flat, pos_block)


# speedup vs baseline: 1.6587x; 1.6587x over previous
import functools

import jax
import jax.numpy as jnp
from jax import lax
from jax.experimental import pallas as pl
from jax.experimental.pallas import tpu as pltpu
from jax.experimental.pallas import tpu_sc as plsc

VOCAB = 100000
HIDDEN = 128
B = 1024
L = 50

NC = 2
NS = 16
NW = NC * NS
LANES = 16

N_SPLITS = 2
B_HALF = B // N_SPLITS
TOTAL_H = B_HALF * L
PER_W = TOTAL_H // NW
SEQ_PER_CHUNK = 4
CHUNK = SEQ_PER_CHUNK * L
N_CHUNKS = PER_W // CHUNK


def _sc_embed_half(ids_half, token_embedding, pos_block):
    mesh = plsc.VectorSubcoreMesh(core_axis_name="c", subcore_axis_name="s")

    @functools.partial(
        pl.kernel,
        out_type=jax.ShapeDtypeStruct((TOTAL_H, HIDDEN), jnp.float32),
        mesh=mesh,
        scratch_types=[
            pltpu.VMEM((PER_W,), jnp.int32),
            pltpu.VMEM((CHUNK, HIDDEN), jnp.float32),
            pltpu.VMEM((CHUNK, HIDDEN), jnp.float32),
            pltpu.VMEM((L, HIDDEN), jnp.float32),
            pltpu.SemaphoreType.DMA,
            pltpu.SemaphoreType.DMA,
            pltpu.SemaphoreType.DMA,
            pltpu.SemaphoreType.DMA,
        ],
    )
    def k(ids_hbm, table_hbm, pos_hbm, out_hbm,
          idx_v, rows0, rows1, pos_v, gsem0, gsem1, osem0, osem1):
        wid = lax.axis_index("s") * NC + lax.axis_index("c")
        base = wid * PER_W
        pltpu.sync_copy(ids_hbm.at[pl.ds(base, PER_W)], idx_v)
        pltpu.sync_copy(pos_hbm, pos_v)

        rows = (rows0, rows1)
        gsems = (gsem0, gsem1)
        osems = (osem0, osem1)

        def add_pos(rv):
            @pl.loop(0, L)
            def _(l):
                for c1 in range(0, HIDDEN, LANES):
                    p = pos_v.at[pl.ds(l, 1), pl.ds(c1, LANES)][...]
                    for s in range(SEQ_PER_CHUNK):
                        slc = (pl.ds(s * L + l, 1), pl.ds(c1, LANES))
                        rv.at[*slc][...] = rv.at[*slc][...] + p

        def start_gather(g):
            return pltpu.async_copy(
                table_hbm.at[idx_v.at[pl.ds(g * CHUNK, CHUNK)]],
                rows[g % 2], gsems[g % 2])

        def start_out(g):
            return pltpu.async_copy(
                rows[g % 2], out_hbm.at[pl.ds(base + g * CHUNK, CHUNK)],
                osems[g % 2])

        gcp = [None] * N_CHUNKS
        ocp = [None] * N_CHUNKS
        gcp[0] = start_gather(0)
        for g in range(N_CHUNKS):
            if g + 1 < N_CHUNKS:
                if g + 1 >= 2:
                    ocp[g - 1].wait()
                gcp[g + 1] = start_gather(g + 1)
            gcp[g].wait()
            add_pos(rows[g % 2])
            ocp[g] = start_out(g)
        ocp[N_CHUNKS - 2].wait()
        ocp[N_CHUNKS - 1].wait()

    return k(ids_half, token_embedding, pos_block)


def kernel(input_ids, token_embedding, position_embedding):
    ids_flat = input_ids.reshape(B * L).astype(jnp.int32)
    pos_block = position_embedding[0, :L, :]
    halves = []
    for s in range(N_SPLITS):
        h = _sc_embed_half(ids_flat[s * TOTAL_H:(s + 1) * TOTAL_H],
                           token_embedding, pos_block)
        halves.append(h.reshape(B_HALF, L, HIDDEN))
    return jnp.concatenate(halves, axis=0)


# speedup vs baseline: 1.9524x; 1.1771x over previous
import functools

import jax
import jax.numpy as jnp
from jax import lax
from jax.experimental import pallas as pl
from jax.experimental.pallas import tpu as pltpu
from jax.experimental.pallas import tpu_sc as plsc

VOCAB = 100000
HIDDEN = 128
B = 1024
L = 50

NC = 2
NS = 16
NW = NC * NS
LANES = 16

TOTAL = B * L
PER_W = TOTAL // NW
SEQ_PER_CHUNK = 8
CHUNK = SEQ_PER_CHUNK * L
N_CHUNKS = PER_W // CHUNK


def _sc_embed(ids_flat, token_embedding, pos_block):
    mesh = plsc.VectorSubcoreMesh(core_axis_name="c", subcore_axis_name="s")

    @functools.partial(
        pl.kernel,
        out_type=jax.ShapeDtypeStruct((TOTAL, HIDDEN), jnp.float32),
        mesh=mesh,
        scratch_types=[
            pltpu.VMEM((PER_W,), jnp.int32),
            pltpu.VMEM((CHUNK, HIDDEN), jnp.float32),
            pltpu.VMEM((CHUNK, HIDDEN), jnp.float32),
            pltpu.VMEM((L, HIDDEN), jnp.float32),
            pltpu.SemaphoreType.DMA,
            pltpu.SemaphoreType.DMA,
            pltpu.SemaphoreType.DMA,
            pltpu.SemaphoreType.DMA,
        ],
    )
    def k(ids_hbm, table_hbm, pos_hbm, out_hbm,
          idx_v, rows0, rows1, pos_v, gsem0, gsem1, osem0, osem1):
        wid = lax.axis_index("s") * NC + lax.axis_index("c")
        base = wid * PER_W
        pltpu.sync_copy(ids_hbm.at[pl.ds(base, PER_W)], idx_v)
        pltpu.sync_copy(pos_hbm, pos_v)

        rows = (rows0, rows1)
        gsems = (gsem0, gsem1)
        osems = (osem0, osem1)

        def add_pos(rv):
            @pl.loop(0, L)
            def _(l):
                for c1 in range(0, HIDDEN, LANES):
                    p = pos_v.at[pl.ds(l, 1), pl.ds(c1, LANES)][...]
                    for s in range(SEQ_PER_CHUNK):
                        slc = (pl.ds(s * L + l, 1), pl.ds(c1, LANES))
                        rv.at[*slc][...] = rv.at[*slc][...] + p

        def start_gather(g):
            return pltpu.async_copy(
                table_hbm.at[idx_v.at[pl.ds(g * CHUNK, CHUNK)]],
                rows[g % 2], gsems[g % 2])

        def start_out(g):
            return pltpu.async_copy(
                rows[g % 2], out_hbm.at[pl.ds(base + g * CHUNK, CHUNK)],
                osems[g % 2])

        gcp = [None] * N_CHUNKS
        ocp = [None] * N_CHUNKS
        gcp[0] = start_gather(0)
        for g in range(N_CHUNKS):
            if g + 1 < N_CHUNKS:
                if g + 1 >= 2:
                    ocp[g - 1].wait()
                gcp[g + 1] = start_gather(g + 1)
            gcp[g].wait()
            add_pos(rows[g % 2])
            ocp[g] = start_out(g)
        ocp[N_CHUNKS - 2].wait()
        ocp[N_CHUNKS - 1].wait()

    return k(ids_flat, token_embedding, pos_block)


def kernel(input_ids, token_embedding, position_embedding):
    ids_flat = input_ids.reshape(TOTAL).astype(jnp.int32)
    pos_block = position_embedding[0, :L, :]
    out = _sc_embed(ids_flat, token_embedding, pos_block)
    return out.reshape(B, L, HIDDEN)
